# Initial kernel scaffold; baseline (speedup 1.0000x reference)
import jax
import jax.numpy as jnp
from jax.experimental import pallas as pl


def kernel(indices, offsets, table):
    raise NotImplementedError("write your pallas kernel here")


# speedup vs baseline: 113.7626x; 113.7626x over previous
import functools

import jax
import jax.numpy as jnp
from jax import lax
from jax.experimental import pallas as pl
from jax.experimental.pallas import tpu as pltpu
from jax.experimental.pallas import tpu_sc as plsc

VOCAB = 1000000
D = 32
BATCH = 16384
TOTAL = 819200

NTILES = 16
NW = 2 * NTILES
BAGS_W = BATCH // NW
CHUNK = 128
ACC_ROWS = BAGS_W + 8
DUMP = BAGS_W
SEARCH_STEPS = 10


def _emb_body(indices_hbm, offsets_hbm, table_hbm, out_hbm,
              offs_v, idx_v, rows_v, tmp_v, acc_v, sem):
    c = lax.axis_index("c")
    t = lax.axis_index("s")
    w = c * NTILES + t
    base_bag = w * BAGS_W
    lane = lax.iota(jnp.int32, 16)

    pltpu.sync_copy(offsets_hbm.at[pl.ds(base_bag, BAGS_W)],
                    offs_v.at[pl.ds(0, BAGS_W)])
    offs_v[pl.ds(BAGS_W, 16)] = jnp.full((16,), jnp.iinfo(jnp.int32).max,
                                         jnp.int32)
    o_start = offs_v[pl.ds(0, 16)][0]

    end_pos = jnp.where(w == NW - 1, BATCH - 16, base_bag + BAGS_W)
    pltpu.sync_copy(offsets_hbm.at[pl.ds(end_pos, 16)], tmp_v)
    o_end = jnp.where(w == NW - 1, TOTAL, tmp_v[...][0])

    def _zero(i, carry):
        acc_v[i // 2, pl.ds((i % 2) * 16, 16)] = jnp.zeros((16,), jnp.float32)
        return carry
    lax.fori_loop(0, 2 * ACC_ROWS, _zero, 0)

    start_al = (o_start // CHUNK) * CHUNK
    end_al = ((o_end + CHUNK - 1) // CHUNK) * CHUNK
    n_chunks = (end_al - start_al) // CHUNK

    def _chunk(k, carry):
        pstart = start_al + k * CHUNK
        pltpu.sync_copy(indices_hbm.at[pl.ds(pstart, CHUNK)], idx_v)
        gather = pltpu.async_copy(table_hbm.at[idx_v], rows_v, sem)

        segs = []
        for g in range(CHUNK // 16):
            p = pstart + g * 16 + lane
            lo_v = jnp.zeros((16,), jnp.int32)
            hi_v = jnp.full((16,), BAGS_W, jnp.int32)
            for _ in range(SEARCH_STEPS):
                mid = (lo_v + hi_v) // 2
                ov = plsc.load_gather(offs_v, [mid])
                take_hi = ov <= p
                lo_v = jnp.where(take_hi, mid + 1, lo_v)
                hi_v = jnp.where(take_hi, hi_v, mid)
            seg = lo_v - 1
            in_range = (p >= o_start) & (p < o_end)
            segs.append(jnp.where(in_range, seg, DUMP))

        gather.wait()
        for g in range(CHUNK // 16):
            row_idx = g * 16 + lane
            for col in range(D):
                col_idx = (lane + col) & (D - 1)
                vals = plsc.load_gather(rows_v, [row_idx, col_idx])
                plsc.addupdate_scatter(acc_v, [segs[g], col_idx], vals)
        return carry

    lax.fori_loop(0, n_chunks, _chunk, 0)

    pltpu.sync_copy(acc_v.at[pl.ds(0, BAGS_W)],
                    out_hbm.at[pl.ds(base_bag, BAGS_W)])


_emb_kernel = functools.partial(
    pl.kernel,
    out_type=jax.ShapeDtypeStruct((BATCH, D), jnp.float32),
    mesh=plsc.VectorSubcoreMesh(core_axis_name="c", subcore_axis_name="s"),
    compiler_params=pltpu.CompilerParams(
        needs_layout_passes=False, use_tc_tiling_on_sc=False),
    scratch_types=[
        pltpu.VMEM((BAGS_W + 16,), jnp.int32),
        pltpu.VMEM((CHUNK,), jnp.int32),
        pltpu.VMEM((CHUNK, D), jnp.float32),
        pltpu.VMEM((16,), jnp.int32),
        pltpu.VMEM((ACC_ROWS, D), jnp.float32),
        pltpu.SemaphoreType.DMA,
    ],
)(_emb_body)


@jax.jit
def kernel(indices, offsets, table):
    return _emb_kernel(indices, offsets, table)


# speedup vs baseline: 124.2164x; 1.0919x over previous
import functools

import jax
import jax.numpy as jnp
from jax import lax
from jax.experimental import pallas as pl
from jax.experimental.pallas import tpu as pltpu
from jax.experimental.pallas import tpu_sc as plsc

VOCAB = 1000000
D = 32
BATCH = 16384
TOTAL = 819200

NTILES = 16
NW = 2 * NTILES
BAGS_W = BATCH // NW
CHUNK = 128
ACC_ROWS = BAGS_W + 8
DUMP = BAGS_W
SEARCH_STEPS = 10


def _emb_body(indices_hbm, offsets_hbm, table_hbm, out_hbm,
              offs_v, idx_v0, idx_v1, rows_v0, rows_v1, tmp_v, acc_v,
              sem0, sem1):
    c = lax.axis_index("c")
    t = lax.axis_index("s")
    w = c * NTILES + t
    base_bag = w * BAGS_W
    lane = lax.iota(jnp.int32, 16)

    pltpu.sync_copy(offsets_hbm.at[pl.ds(base_bag, BAGS_W)],
                    offs_v.at[pl.ds(0, BAGS_W)])
    offs_v[pl.ds(BAGS_W, 16)] = jnp.full((16,), jnp.iinfo(jnp.int32).max,
                                         jnp.int32)
    o_start = offs_v[pl.ds(0, 16)][0]

    end_pos = jnp.where(w == NW - 1, BATCH - 16, base_bag + BAGS_W)
    pltpu.sync_copy(offsets_hbm.at[pl.ds(end_pos, 16)], tmp_v)
    o_end = jnp.where(w == NW - 1, TOTAL, tmp_v[...][0])

    def _zero(i, carry):
        acc_v[i // 2, pl.ds((i % 2) * 16, 16)] = jnp.zeros((16,), jnp.float32)
        return carry
    lax.fori_loop(0, 2 * ACC_ROWS, _zero, 0)

    start_al = (o_start // CHUNK) * CHUNK
    end_al = ((o_end + CHUNK - 1) // CHUNK) * CHUNK
    n_chunks = (end_al - start_al) // CHUNK

    idx_b = (idx_v0, idx_v1)
    rows_b = (rows_v0, rows_v1)
    sem_b = (sem0, sem1)

    def _fire(kid, b):
        pstart = start_al + kid * CHUNK
        pltpu.sync_copy(indices_hbm.at[pl.ds(pstart, CHUNK)], idx_b[b])
        pltpu.async_copy(table_hbm.at[idx_b[b]], rows_b[b], sem_b[b])

    def _process(kid, b):
        pstart = start_al + kid * CHUNK

        segs = []
        for g in range(CHUNK // 16):
            p = pstart + g * 16 + lane
            lo_v = jnp.zeros((16,), jnp.int32)
            hi_v = jnp.full((16,), BAGS_W, jnp.int32)
            for _ in range(SEARCH_STEPS):
                mid = (lo_v + hi_v) // 2
                ov = plsc.load_gather(offs_v, [mid])
                take_hi = ov <= p
                lo_v = jnp.where(take_hi, mid + 1, lo_v)
                hi_v = jnp.where(take_hi, hi_v, mid)
            seg = lo_v - 1
            in_range = (p >= o_start) & (p < o_end)
            segs.append(jnp.where(in_range, seg, DUMP))

        pltpu.make_async_copy(table_hbm.at[idx_b[b]], rows_b[b],
                              sem_b[b]).wait()
        for g in range(CHUNK // 16):
            row_idx = g * 16 + lane
            for col in range(D):
                col_idx = (lane + col) & (D - 1)
                vals = plsc.load_gather(rows_b[b], [row_idx, col_idx])
                plsc.addupdate_scatter(acc_v, [segs[g], col_idx], vals)

    for b in range(2):
        @pl.when(b < n_chunks)
        def _():
            _fire(b, b)

    def _pair(j, carry):
        for b in range(2):
            kid = 2 * j + b

            @pl.when(kid < n_chunks)
            def _():
                _process(kid, b)

                @pl.when(kid + 2 < n_chunks)
                def _():
                    _fire(kid + 2, b)
        return carry

    lax.fori_loop(0, (n_chunks + 1) // 2, _pair, 0)

    pltpu.sync_copy(acc_v.at[pl.ds(0, BAGS_W)],
                    out_hbm.at[pl.ds(base_bag, BAGS_W)])


_emb_kernel = functools.partial(
    pl.kernel,
    out_type=jax.ShapeDtypeStruct((BATCH, D), jnp.float32),
    mesh=plsc.VectorSubcoreMesh(core_axis_name="c", subcore_axis_name="s"),
    compiler_params=pltpu.CompilerParams(
        needs_layout_passes=False, use_tc_tiling_on_sc=False),
    scratch_types=[
        pltpu.VMEM((BAGS_W + 16,), jnp.int32),
        pltpu.VMEM((CHUNK,), jnp.int32),
        pltpu.VMEM((CHUNK,), jnp.int32),
        pltpu.VMEM((CHUNK, D), jnp.float32),
        pltpu.VMEM((CHUNK, D), jnp.float32),
        pltpu.VMEM((16,), jnp.int32),
        pltpu.VMEM((ACC_ROWS, D), jnp.float32),
        pltpu.SemaphoreType.DMA,
        pltpu.SemaphoreType.DMA,
    ],
)(_emb_body)


@jax.jit
def kernel(indices, offsets, table):
    return _emb_kernel(indices, offsets, table)
